# Initial kernel scaffold; baseline (speedup 1.0000x reference)
import jax
import jax.numpy as jnp
from jax.experimental import pallas as pl


def kernel(input):
    raise NotImplementedError("write your pallas kernel here")


# speedup vs baseline: 4.1249x; 4.1249x over previous
import functools

import jax
import jax.numpy as jnp
from jax import lax
from jax.experimental import pallas as pl
from jax.experimental.pallas import tpu as pltpu
from jax.experimental.pallas import tpu_sc as plsc

R = 64
N = 8192
TOPK = 128
NC = 2
NS = 16
L = 16
NW = NC * NS
ROWS_PER_W = R // NW
NV_FULL = N // L

import numpy as np

MININT = np.int32(-2147483648)
POSMASK = np.int32(0x7FFFFFFF)


def _sc_topk_body(x_hbm, ov_hbm, oi_hbm, row_v, bak, bai, bbk, bbi, wk, wi, wv):
    wid = lax.axis_index("c") * NS + lax.axis_index("s")
    lane = lax.iota(jnp.int32, L)
    zero_v = jnp.zeros((L,), jnp.int32)

    def do_row(r, _unused):
        row = wid * ROWS_PER_W + r
        pltpu.sync_copy(x_hbm.at[pl.ds(row * N, N)], row_v)

        def make_append(dst_k, dst_i):
            def append(src_base, count, woffv):
                trips = (count + L - 1) >> 4

                def body(i, _):
                    kv = dst_k[pl.ds(src_base + i * L, L)]
                    iv = dst_i[pl.ds(src_base + i * L, L)]
                    off = i * L + lane
                    valid = off < count
                    pos = woffv + off
                    plsc.store_scatter(wk, [pos], kv, mask=valid)
                    plsc.store_scatter(wi, [pos], iv, mask=valid)
                    return 0

                lax.fori_loop(0, trips, body, 0)
                return woffv + count

            return append

        def decide(dst_k, dst_i, cnt_hi, cnt_lo, state, is_last):
            n, k, base, woffv, done = state
            append = make_append(dst_k, dst_i)
            keep_hi = cnt_hi >= k
            app1 = jnp.logical_and(jnp.logical_not(done),
                                   jnp.logical_not(keep_hi))
            woffv = append(jnp.int32(0), jnp.where(app1, cnt_hi, 0), woffv)
            n2 = jnp.where(keep_hi, cnt_hi, cnt_lo)
            k2 = jnp.where(keep_hi, k, k - cnt_hi)
            base2 = jnp.where(keep_hi, jnp.int32(0), jnp.int32(N))
            n2 = jnp.where(done, n, n2)
            k2 = jnp.where(done, k, k2)
            base2 = jnp.where(done, base, base2)
            trig = jnp.logical_and(
                jnp.logical_not(done),
                jnp.logical_or(n2 == k2, jnp.bool_(is_last)))
            woffv = append(base2, jnp.where(trig, k2, 0), woffv)
            done2 = jnp.logical_or(done, trig)
            return (n2, k2, base2, woffv, done2)

        def half_pass(bit, srcs, dsts, state, is_last):
            n, k, base, woffv, done = state
            src_k, src_i = srcs
            dst_k, dst_i = dsts
            nv = jnp.where(done, 0, (n + L - 1) >> 4)

            def body(i, carry):
                oh, ol = carry
                kv = src_k[pl.ds(base + i * L, L)]
                iv = src_i[pl.ds(base + i * L, L)]
                off = i * L + lane
                valid = off < n
                bitv = lax.shift_right_logical(
                    kv, jnp.full((L,), bit, jnp.int32)) & 1
                mhi = jnp.logical_and(bitv == 1, valid)
                mlo = jnp.logical_and(bitv == 0, valid)
                cs_hi = plsc.cumsum(mhi.astype(jnp.int32))
                cs_lo = plsc.cumsum(mlo.astype(jnp.int32))
                plsc.store_scatter(dst_k, [oh + cs_hi - 1], kv, mask=mhi)
                plsc.store_scatter(dst_i, [oh + cs_hi - 1], iv, mask=mhi)
                plsc.store_scatter(dst_k, [N + (ol + cs_lo - 1)], kv, mask=mlo)
                plsc.store_scatter(dst_i, [N + (ol + cs_lo - 1)], iv, mask=mlo)
                return (oh + plsc.all_reduce_population_count(mhi),
                        ol + plsc.all_reduce_population_count(mlo))

            oh, ol = lax.fori_loop(0, nv, body, (zero_v, zero_v))
            return decide(dst_k, dst_i, jnp.max(oh), jnp.max(ol), state,
                          is_last)

        def pre_body(i, carry):
            oh, ol = carry
            x = row_v[pl.ds(i * L, L)]
            u = lax.bitcast_convert_type(x, jnp.int32)
            key = u ^ ((u >> 31) | MININT)
            idx = i * L + lane
            mhi = key < 0
            mlo = jnp.logical_not(mhi)
            cs_hi = plsc.cumsum(mhi.astype(jnp.int32))
            cs_lo = plsc.cumsum(mlo.astype(jnp.int32))
            plsc.store_scatter(bbk, [oh + cs_hi - 1], key, mask=mhi)
            plsc.store_scatter(bbi, [oh + cs_hi - 1], idx, mask=mhi)
            plsc.store_scatter(bbk, [N + (ol + cs_lo - 1)], key, mask=mlo)
            plsc.store_scatter(bbi, [N + (ol + cs_lo - 1)], idx, mask=mlo)
            return (oh + plsc.all_reduce_population_count(mhi),
                    ol + plsc.all_reduce_population_count(mlo))

        oh, ol = lax.fori_loop(0, NV_FULL, pre_body, (zero_v, zero_v))
        state = (jnp.int32(N), jnp.int32(TOPK), jnp.int32(0), zero_v,
                 jnp.bool_(False))
        state = decide(bbk, bbi, jnp.max(oh), jnp.max(ol), state, False)

        def outer(j, st):
            b1 = 30 - 2 * j
            st = half_pass(b1, (bbk, bbi), (bak, bai), st, False)
            st = half_pass(b1 - 1, (bak, bai), (bbk, bbi), st, False)
            return st

        state = lax.fori_loop(0, 15, outer, state)
        state = half_pass(0, (bbk, bbi), (bak, bai), state, True)

        def conv(i, _):
            kv = wk[pl.ds(i * L, L)]
            m = jnp.where(kv < 0, MININT, POSMASK)
            wv[pl.ds(i * L, L)] = lax.bitcast_convert_type(kv ^ m, jnp.float32)
            return 0

        lax.fori_loop(0, TOPK // L, conv, 0)
        pltpu.sync_copy(wv, ov_hbm.at[pl.ds(row * TOPK, TOPK)])
        pltpu.sync_copy(wi, oi_hbm.at[pl.ds(row * TOPK, TOPK)])
        return 0

    lax.fori_loop(0, ROWS_PER_W, do_row, 0)


@functools.lru_cache(maxsize=1)
def _build_sc_topk():
    mesh = plsc.VectorSubcoreMesh(core_axis_name="c", subcore_axis_name="s")
    return pl.kernel(
        _sc_topk_body,
        out_type=(jax.ShapeDtypeStruct((R * TOPK,), jnp.float32),
                  jax.ShapeDtypeStruct((R * TOPK,), jnp.int32)),
        mesh=mesh,
        compiler_params=pltpu.CompilerParams(needs_layout_passes=False),
        scratch_types=[
            pltpu.VMEM((N,), jnp.float32),
            pltpu.VMEM((2 * N + L,), jnp.int32),
            pltpu.VMEM((2 * N + L,), jnp.int32),
            pltpu.VMEM((2 * N + L,), jnp.int32),
            pltpu.VMEM((2 * N + L,), jnp.int32),
            pltpu.VMEM((TOPK,), jnp.int32),
            pltpu.VMEM((TOPK,), jnp.int32),
            pltpu.VMEM((TOPK,), jnp.float32),
        ],
    )


def _tc_sort_body(v_ref, i_ref, ov_ref, oi_ref):
    v = v_ref[...]
    ix = i_ref[...]
    va = v[:, :, None]
    vb = v[:, None, :]
    ia = ix[:, :, None]
    ib = ix[:, None, :]
    beats = jnp.logical_or(vb > va, jnp.logical_and(vb == va, ib < ia))
    rank = jnp.sum(beats.astype(jnp.int32), axis=2)
    p = lax.broadcasted_iota(jnp.int32, (R, TOPK, TOPK), 2)
    sel = rank[:, :, None] == p
    ov_ref[...] = jnp.sum(jnp.where(sel, va, 0.0), axis=1)
    oi_ref[...] = jnp.sum(jnp.where(sel, ia, 0), axis=1)


_tc_sort = pl.pallas_call(
    _tc_sort_body,
    out_shape=(jax.ShapeDtypeStruct((R, TOPK), jnp.float32),
               jax.ShapeDtypeStruct((R, TOPK), jnp.int32)),
)


def kernel(input):
    flat = input.reshape(R * N)
    cv, ci = _build_sc_topk()(flat)
    return _tc_sort(cv.reshape(R, TOPK), ci.reshape(R, TOPK))
